# Initial kernel scaffold; baseline (speedup 1.0000x reference)
import jax
import jax.numpy as jnp
from jax.experimental import pallas as pl


def kernel(zs, emb_weight):
    raise NotImplementedError("write your pallas kernel here")


# speedup vs baseline: 6.1177x; 6.1177x over previous
import functools

import jax
import jax.numpy as jnp
from jax import lax
from jax.experimental import pallas as pl
from jax.experimental.pallas import tpu as pltpu
from jax.experimental.pallas import tpu_sc as plsc

_NUM_CORES = 2
_NUM_SUBCORES = 16
_NUM_WORKERS = _NUM_CORES * _NUM_SUBCORES

_CHUNK = 1024


def _make_gather(B, D, chunk):
    per_w = B // _NUM_WORKERS
    n_chunks = per_w // chunk
    assert per_w % chunk == 0 and B % _NUM_WORKERS == 0

    mesh = plsc.VectorSubcoreMesh(core_axis_name="c", subcore_axis_name="s")

    @functools.partial(
        pl.kernel,
        out_type=jax.ShapeDtypeStruct((B, D), jnp.float32),
        mesh=mesh,
        scratch_types=[
            pltpu.VMEM((chunk,), jnp.int32),
            pltpu.VMEM((chunk, D), jnp.float32),
            pltpu.SemaphoreType.DMA,
        ],
        compiler_params=pltpu.CompilerParams(use_tc_tiling_on_sc=False),
    )
    def k(zs_hbm, table_hbm, out_hbm, idx_v, rows_v, sem):
        wid = lax.axis_index("s") * _NUM_CORES + lax.axis_index("c")
        base = wid * per_w

        @pl.loop(0, n_chunks)
        def _(j):
            off = base + j * chunk
            pltpu.sync_copy(zs_hbm.at[pl.ds(off, chunk)], idx_v)
            pltpu.async_copy(table_hbm.at[idx_v], rows_v, sem).wait()
            pltpu.sync_copy(rows_v, out_hbm.at[pl.ds(off, chunk)])

    return k


def kernel(zs, emb_weight):
    batch, seq = zs.shape
    d = emb_weight.shape[1]
    flat = zs.reshape(-1).astype(jnp.int32)
    out = _make_gather(flat.shape[0], d, _CHUNK)(flat, emb_weight)
    return out.reshape(batch, seq, d)


# speedup vs baseline: 6.3981x; 1.0458x over previous
import functools

import jax
import jax.numpy as jnp
from jax import lax
from jax.experimental import pallas as pl
from jax.experimental.pallas import tpu as pltpu
from jax.experimental.pallas import tpu_sc as plsc

_NUM_CORES = 2
_NUM_SUBCORES = 16
_NUM_WORKERS = _NUM_CORES * _NUM_SUBCORES

_CHUNK_ROWS = 8
_NBUF = 2


def _make_gather(batch, seq, D, chunk_rows, nbuf):
    rows_w = batch // _NUM_WORKERS
    n_chunks = rows_w // chunk_rows
    assert batch % _NUM_WORKERS == 0 and rows_w % chunk_rows == 0
    assert n_chunks % nbuf == 0 and n_chunks >= 2 * nbuf

    mesh = plsc.VectorSubcoreMesh(core_axis_name="c", subcore_axis_name="s")

    @functools.partial(
        pl.kernel,
        out_type=jax.ShapeDtypeStruct((batch, seq, D), jnp.float32),
        mesh=mesh,
        scratch_types=[
            [pltpu.VMEM((chunk_rows, seq), jnp.int32) for _ in range(nbuf)],
            [pltpu.VMEM((chunk_rows, seq, D), jnp.float32) for _ in range(nbuf)],
            [pltpu.SemaphoreType.DMA for _ in range(nbuf)],
            [pltpu.SemaphoreType.DMA for _ in range(nbuf)],
        ],
        compiler_params=pltpu.CompilerParams(use_tc_tiling_on_sc=False),
    )
    def k(zs_hbm, table_hbm, out_hbm, idx_v, rows_v, gsem, wsem):
        wid = lax.axis_index("s") * _NUM_CORES + lax.axis_index("c")
        base = wid * rows_w

        def start_gather(j, s):
            r0 = base + j * chunk_rows
            pltpu.sync_copy(zs_hbm.at[pl.ds(r0, chunk_rows), :], idx_v[s])
            for r in range(chunk_rows):
                pltpu.async_copy(table_hbm.at[idx_v[s].at[r]], rows_v[s].at[r], gsem[s])

        def start_wb(j, s):
            r0 = base + j * chunk_rows
            pltpu.async_copy(rows_v[s], out_hbm.at[pl.ds(r0, chunk_rows)], wsem[s])

        def wait(sem, s):
            pltpu.make_async_copy(out_hbm.at[pl.ds(base, chunk_rows)], rows_v[s], sem).wait()

        for s in range(nbuf):
            start_gather(s, s)
        for s in range(nbuf - 1):
            wait(gsem[s], s)
            start_wb(s, s)

        @pl.loop(nbuf, n_chunks, step=nbuf)
        def _(j0):
            for s in range(nbuf):
                jj = j0 + s
                p = (s - 1) % nbuf
                wait(wsem[s], s)
                start_gather(jj, s)
                wait(gsem[p], p)
                start_wb(jj - 1, p)

        wait(gsem[nbuf - 1], nbuf - 1)
        start_wb(n_chunks - 1, nbuf - 1)
        for s in range(nbuf):
            wait(wsem[s], s)

    return k


def kernel(zs, emb_weight):
    batch, seq = zs.shape
    d = emb_weight.shape[1]
    made = _make_gather(batch, seq, d, _CHUNK_ROWS, _NBUF)
    return made(zs.astype(jnp.int32), emb_weight)
